# baseline (speedup 1.0000x reference)
import functools

import jax
import jax.numpy as jnp
from jax import lax
from jax.experimental import pallas as pl
from jax.experimental.pallas import tpu as pltpu
from jax.experimental.pallas import tpu_sc as plsc

_K = 16
_GS = 8
_NG = 1024
_QT = 256
_NT = 512
_BIG = 3.0e38
_IBIG = 2**31 - 1

_SC_CORES = 2
_SC_SUBCORES = 16
_NWORK = _SC_CORES * _SC_SUBCORES
_RCH = 512


def _leaky(x):
    return jnp.where(x > 0, x, 0.1 * x)


_HP = dict(preferred_element_type=jnp.float32, precision=lax.Precision.HIGHEST)


def _topk_body(a2_ref, b1_ref, grp_ref, d_ref):
    lhs = a2_ref[:, :].astype(jnp.bfloat16)
    rhs = b1_ref[:, :].astype(jnp.bfloat16)
    d0 = lax.dot_general(lhs, rhs, (((1,), (0,)), ((), ())),
                         preferred_element_type=jnp.float32)
    x, y, z = b1_ref[0:1, :], b1_ref[1:2, :], b1_ref[2:3, :]
    rn = (x * x + y * y) + z * z
    cn = a2_ref[:, 3:4]
    d_ref[:, :] = (d0 + rn) + cn
    g = d_ref[pl.ds(0, _NG), :]
    for c in range(1, _GS):
        g = jnp.minimum(g, d_ref[pl.ds(c * _NG, _NG), :])
    gio = lax.broadcasted_iota(jnp.int32, (_NG, _QT), 0)
    vprev = jnp.full((_QT,), -_BIG, jnp.float32)
    iprev = jnp.full((_QT,), -1, jnp.int32)
    for k in range(_K):
        keep = (g > vprev[None, :]) | ((g == vprev[None, :])
                                       & (gio > iprev[None, :]))
        dv = jnp.where(keep, g, _BIG)
        m = jnp.min(dv, axis=0)
        ii = jnp.min(jnp.where(dv == m[None, :], gio, _IBIG), axis=0)
        vprev, iprev = m, ii
        grp_ref[k, :] = ii


def _topk(a2p, b1m):
    B, n2, _ = a2p.shape
    n1 = b1m.shape[2]
    return pl.pallas_call(
        _topk_body,
        grid=(B, n1 // _QT),
        in_specs=[
            pl.BlockSpec((None, n2, 8), lambda b, t: (b, 0, 0)),
            pl.BlockSpec((None, 8, _QT), lambda b, t: (b, 0, t)),
        ],
        out_specs=pl.BlockSpec((None, _K, _QT), lambda b, t: (b, 0, t)),
        out_shape=jax.ShapeDtypeStruct((B, _K, n1), jnp.int32),
        scratch_shapes=[pltpu.VMEM((n2, _QT), jnp.float32)],
    )(a2p, b1m)


def _refine(grp, kx, ky, kz, cn, qx, qy, qz, rn, B, n1, n2):
    K = _K
    wpb = _NWORK // B
    npb = n1 // wpb
    npacks = npb // 16
    mesh = plsc.VectorSubcoreMesh(core_axis_name="c", subcore_axis_name="s")

    @functools.partial(
        pl.kernel,
        out_type=jax.ShapeDtypeStruct((B * K * n1,), jnp.int32),
        mesh=mesh,
        compiler_params=pltpu.CompilerParams(needs_layout_passes=False),
        scratch_types=[
            pltpu.VMEM((n2,), jnp.float32),
            pltpu.VMEM((n2,), jnp.float32),
            pltpu.VMEM((n2,), jnp.float32),
            pltpu.VMEM((n2,), jnp.float32),
            pltpu.VMEM((npb,), jnp.float32),
            pltpu.VMEM((npb,), jnp.float32),
            pltpu.VMEM((npb,), jnp.float32),
            pltpu.VMEM((npb,), jnp.float32),
            pltpu.VMEM((K * npb,), jnp.float32),
            pltpu.VMEM((K * _GS * 16,), jnp.float32),
            pltpu.VMEM((K * 16,), jnp.float32),
            pltpu.VMEM((K * 16,), jnp.int32),
            pltpu.VMEM((K * npb,), jnp.int32),
        ],
    )
    def rk(grp_h, kx_h, ky_h, kz_h, cn_h, qx_h, qy_h, qz_h, rn_h, out_h,
           kx_v, ky_v, kz_v, cn_v, qx_v, qy_v, qz_v, rn_v, grp_v,
           cand_d, M_d, M_i, o_w):
        wid = lax.axis_index("s") * _SC_CORES + lax.axis_index("c")
        b = wid // wpb
        n0 = (wid % wpb) * npb
        pltpu.sync_copy(kx_h.at[pl.ds(b * n2, n2)], kx_v)
        pltpu.sync_copy(ky_h.at[pl.ds(b * n2, n2)], ky_v)
        pltpu.sync_copy(kz_h.at[pl.ds(b * n2, n2)], kz_v)
        pltpu.sync_copy(cn_h.at[pl.ds(b * n2, n2)], cn_v)
        pltpu.sync_copy(qx_h.at[pl.ds(b * n1 + n0, npb)], qx_v)
        pltpu.sync_copy(qy_h.at[pl.ds(b * n1 + n0, npb)], qy_v)
        pltpu.sync_copy(qz_h.at[pl.ds(b * n1 + n0, npb)], qz_v)
        pltpu.sync_copy(rn_h.at[pl.ds(b * n1 + n0, npb)], rn_v)
        for t in range(K):
            pltpu.sync_copy(grp_h.at[pl.ds((b * K + t) * n1 + n0, npb)],
                            grp_v.at[pl.ds(t * npb, npb)])
        lanes = lax.iota(jnp.int32, 16)
        zeros = jnp.zeros((16,), jnp.int32)

        def pack_step(p, carry):
            qxl = qx_v[pl.ds(p * 16, 16)]
            qyl = qy_v[pl.ds(p * 16, 16)]
            qzl = qz_v[pl.ds(p * 16, 16)]
            rnl = rn_v[pl.ds(p * 16, 16)]
            col = p * 16 + lanes

            def grp_step(t, c2):
                gvec = plsc.load_gather(grp_v, [t * npb + col]).astype(jnp.int32)
                bd = jnp.full((16,), _BIG, jnp.float32)
                bi = jnp.full((16,), _IBIG, jnp.int32)
                for e in range(_GS):
                    kvec = gvec * _GS + e
                    dx = plsc.load_gather(kx_v, [kvec])
                    dy = plsc.load_gather(ky_v, [kvec])
                    dz = plsc.load_gather(kz_v, [kvec])
                    cc = plsc.load_gather(cn_v, [kvec])
                    d0 = (dx * qxl + dy * qyl) + dz * qzl
                    d = (d0 + rnl) + cc
                    plsc.store_scatter(cand_d, [(t * _GS + e) * 16 + lanes], d)
                    lt = (d < bd) | ((d == bd) & (kvec < bi))
                    bd = jnp.where(lt, d, bd)
                    bi = jnp.where(lt, kvec, bi)
                plsc.store_scatter(M_d, [t * 16 + lanes], bd)
                plsc.store_scatter(M_i, [t * 16 + lanes], bi)
                return c2

            lax.fori_loop(0, K, grp_step, 0)

            def sel_step(k, c2):
                cv = jnp.full((16,), _BIG, jnp.float32)
                ci = jnp.full((16,), _IBIG, jnp.int32)
                wv = jnp.zeros((16,), jnp.int32)
                for rb in range(K):
                    md = M_d[pl.ds(rb * 16, 16)]
                    mi = M_i[pl.ds(rb * 16, 16)]
                    lt = (md < cv) | ((md == cv) & (mi < ci))
                    cv = jnp.where(lt, md, cv)
                    ci = jnp.where(lt, mi, ci)
                    wv = jnp.where(lt, zeros + rb, wv)
                plsc.store_scatter(o_w, [k * npb + col], ci)
                gw = plsc.load_gather(grp_v, [wv * npb + col]).astype(jnp.int32)
                nd = jnp.full((16,), _BIG, jnp.float32)
                ni = jnp.full((16,), _IBIG, jnp.int32)
                for e in range(_GS):
                    dd = plsc.load_gather(cand_d, [(wv * _GS + e) * 16 + lanes])
                    ii = gw * _GS + e
                    valid = (dd > cv) | ((dd == cv) & (ii > ci))
                    dd2 = jnp.where(valid, dd, _BIG)
                    ii2 = jnp.where(valid, ii, _IBIG)
                    lt = (dd2 < nd) | ((dd2 == nd) & (ii2 < ni))
                    nd = jnp.where(lt, dd2, nd)
                    ni = jnp.where(lt, ii2, ni)
                plsc.store_scatter(M_d, [wv * 16 + lanes], nd)
                plsc.store_scatter(M_i, [wv * 16 + lanes], ni)
                return c2

            lax.fori_loop(0, K, sel_step, 0)
            return carry

        lax.fori_loop(0, npacks, pack_step, 0)
        for t in range(K):
            pltpu.sync_copy(o_w.at[pl.ds(t * npb, npb)],
                            out_h.at[pl.ds((b * K + t) * n1 + n0, npb)])

    return rk(grp, kx, ky, kz, cn, qx, qy, qz, rn)


def _pre_body(f1_ref, f2_ref, p1_ref, p2_ref, w1a_ref, w1b_ref, w1c_ref,
              b1_ref, q_ref, g2_ref):
    dn = (((0,), (1,)), ((), ()))
    g2 = (lax.dot_general(f2_ref[:, :], w1b_ref[:, :], dn, **_HP)
          + lax.dot_general(p2_ref[:, :], w1c_ref[:, :], dn, **_HP))
    g2_ref[:, 0:64] = g2
    g2_ref[:, 64:128] = jnp.zeros_like(g2)
    q_ref[:, :] = (lax.dot_general(f1_ref[:, :], w1a_ref[:, :], dn, **_HP)
                   - lax.dot_general(p1_ref[:, :], w1c_ref[:, :], dn, **_HP)
                   + b1_ref[:, :])


def _pre(feat1, feat2, pc1p, pc2p, w1a, w1b, w1c, b1m):
    B, D, n1 = feat1.shape
    n2 = feat2.shape[2]
    return pl.pallas_call(
        _pre_body,
        grid=(B, n1 // _NT),
        in_specs=[
            pl.BlockSpec((None, D, _NT), lambda b, t: (b, 0, t)),
            pl.BlockSpec((None, D, _NT), lambda b, t: (b, 0, t)),
            pl.BlockSpec((None, 8, _NT), lambda b, t: (b, 0, t)),
            pl.BlockSpec((None, 8, _NT), lambda b, t: (b, 0, t)),
            pl.BlockSpec((D, D), lambda b, t: (0, 0)),
            pl.BlockSpec((D, D), lambda b, t: (0, 0)),
            pl.BlockSpec((D, 8), lambda b, t: (0, 0)),
            pl.BlockSpec((1, D), lambda b, t: (0, 0)),
        ],
        out_specs=[
            pl.BlockSpec((None, _NT, D), lambda b, t: (b, t, 0)),
            pl.BlockSpec((None, _NT, 2 * D), lambda b, t: (b, t, 0)),
        ],
        out_shape=[jax.ShapeDtypeStruct((B, n1, D), jnp.float32),
                   jax.ShapeDtypeStruct((B, n2, 2 * D), jnp.float32)],
    )(feat1, feat2, pc1p, pc2p, w1a, w1b, w1c, b1m)


def _gather(table, fidx):
    rows, d = fidx.shape[0], table.shape[1]
    per_w = rows // _NWORK
    nch = per_w // _RCH
    mesh = plsc.VectorSubcoreMesh(core_axis_name="c", subcore_axis_name="s")

    @functools.partial(
        pl.kernel,
        out_type=jax.ShapeDtypeStruct((rows, d), jnp.float32),
        mesh=mesh,
        scratch_types=[
            pltpu.VMEM((_RCH,), jnp.int32),
            pltpu.VMEM((_RCH, d), jnp.float32),
            pltpu.SemaphoreType.DMA,
        ],
    )
    def gk(table_hbm, idx_hbm, out_hbm, idx_v, rows_v, sem):
        wid = lax.axis_index("s") * _SC_CORES + lax.axis_index("c")

        def step(j, carry):
            base = wid * per_w + j * _RCH
            pltpu.sync_copy(idx_hbm.at[pl.ds(base, _RCH)], idx_v)
            pltpu.async_copy(table_hbm.at[idx_v], rows_v, sem).wait()
            pltpu.sync_copy(rows_v, out_hbm.at[pl.ds(base, _RCH)])
            return carry

        lax.fori_loop(0, nch, step, 0)

    return gk(table, fidx)


def _mlp_body(g_ref, q_ref, w2_ref, b2_ref, o_ref):
    q = q_ref[:, :]
    acc = jnp.full(q.shape, -_BIG, jnp.float32)
    dn = (((1,), (1,)), ((), ()))
    for k in range(_K):
        h = _leaky(g_ref[k, :, 0:64] + q)
        y = _leaky(lax.dot_general(h, w2_ref[:, :], dn, **_HP) + b2_ref[:, :])
        acc = jnp.maximum(acc, y)
    o_ref[:, :] = acc


def _mlp(gath, q, W2, b2m):
    B, K, n1, Dp = gath.shape
    D = q.shape[2]
    return pl.pallas_call(
        _mlp_body,
        grid=(B, n1 // _NT),
        in_specs=[
            pl.BlockSpec((None, K, _NT, Dp), lambda b, t: (b, 0, t, 0)),
            pl.BlockSpec((None, _NT, D), lambda b, t: (b, t, 0)),
            pl.BlockSpec((D, D), lambda b, t: (0, 0)),
            pl.BlockSpec((1, D), lambda b, t: (0, 0)),
        ],
        out_specs=pl.BlockSpec((None, _NT, D), lambda b, t: (b, t, 0)),
        out_shape=jax.ShapeDtypeStruct((B, n1, D), jnp.float32),
    )(gath, q, W2, b2m)


def _bf(x):
    u = lax.bitcast_convert_type(x, jnp.uint32)
    r = (u + jnp.uint32(0x7FFF) + ((u >> 16) & jnp.uint32(1))) \
        & jnp.uint32(0xFFFF0000)
    return lax.bitcast_convert_type(r, jnp.float32)


def kernel(pc1, pc2, feat1, feat2, W1, b1, W2, b2):
    B, _, n1 = pc1.shape
    n2 = pc2.shape[2]
    D = feat1.shape[1]
    f32 = jnp.float32

    xyz2t = jnp.transpose(pc2, (0, 2, 1))
    a2 = jnp.concatenate(
        [-2.0 * xyz2t,
         jnp.sum(xyz2t ** 2, axis=-1, keepdims=True),
         jnp.zeros((B, n2, 4), f32)], axis=-1)
    a2p = a2.reshape(B, _NG, _GS, 8).transpose(0, 2, 1, 3).reshape(B, n2, 8)
    b1m = jnp.concatenate([pc1, jnp.zeros((B, 5, n1), f32)], axis=1)
    grp = _topk(a2p, b1m)

    kx, ky, kz = _bf(-2.0 * pc2[:, 0]), _bf(-2.0 * pc2[:, 1]), _bf(-2.0 * pc2[:, 2])
    cn2 = jnp.sum(xyz2t ** 2, axis=-1)
    qx, qy, qz = _bf(pc1[:, 0]), _bf(pc1[:, 1]), _bf(pc1[:, 2])
    rn1 = (pc1[:, 0] * pc1[:, 0] + pc1[:, 1] * pc1[:, 1]) + pc1[:, 2] * pc1[:, 2]
    knn_f = _refine(grp.astype(f32).reshape(-1), kx.reshape(-1),
                    ky.reshape(-1), kz.reshape(-1), cn2.reshape(-1),
                    qx.reshape(-1), qy.reshape(-1), qz.reshape(-1),
                    rn1.reshape(-1), B, n1, n2)
    knn = knn_f.reshape(B, _K, n1)

    pc1p = jnp.concatenate([pc1, jnp.zeros((B, 5, n1), f32)], axis=1)
    pc2p = jnp.concatenate([pc2, jnp.zeros((B, 5, n2), f32)], axis=1)
    w1a = W1[:, :D]
    w1b = W1[:, D:2 * D]
    w1c = jnp.concatenate([W1[:, 2 * D:], jnp.zeros((D, 5), f32)], axis=1)
    q, g2 = _pre(feat1, feat2, pc1p, pc2p, w1a, w1b, w1c, b1.reshape(1, D))

    fidx = (knn + (jnp.arange(B, dtype=jnp.int32) * n2)[:, None, None]
            ).reshape(-1)
    gath = _gather(g2.reshape(B * n2, 2 * D), fidx).reshape(B, _K, n1, 2 * D)
    out = _mlp(gath, q, W2, b2.reshape(1, D))
    return jnp.transpose(out, (0, 2, 1))


# speedup vs baseline: 1.0819x; 1.0819x over previous
import functools

import jax
import jax.numpy as jnp
from jax import lax
from jax.experimental import pallas as pl
from jax.experimental.pallas import tpu as pltpu
from jax.experimental.pallas import tpu_sc as plsc

_K = 16
_GS = 16
_NG = 512
_QT = 256
_NT = 512
_BIG = 3.0e38
_IBIG = 2**31 - 1

_SC_CORES = 2
_SC_SUBCORES = 16
_NWORK = _SC_CORES * _SC_SUBCORES
_RCH = 512


def _leaky(x):
    return jnp.where(x > 0, x, 0.1 * x)


_HP = dict(preferred_element_type=jnp.float32, precision=lax.Precision.HIGHEST)


def _topk_body(a2_ref, b1_ref, grp_ref, d_ref):
    lhs = a2_ref[:, :].astype(jnp.bfloat16)
    rhs = b1_ref[:, :].astype(jnp.bfloat16)
    d0 = lax.dot_general(lhs, rhs, (((1,), (0,)), ((), ())),
                         preferred_element_type=jnp.float32)
    x, y, z = b1_ref[0:1, :], b1_ref[1:2, :], b1_ref[2:3, :]
    rn = (x * x + y * y) + z * z
    cn = a2_ref[:, 3:4]
    d_ref[:, :] = (d0 + rn) + cn
    g = d_ref[pl.ds(0, _NG), :]
    for c in range(1, _GS):
        g = jnp.minimum(g, d_ref[pl.ds(c * _NG, _NG), :])
    gio = lax.broadcasted_iota(jnp.int32, (_NG, _QT), 0)
    vprev = jnp.full((_QT,), -_BIG, jnp.float32)
    iprev = jnp.full((_QT,), -1, jnp.int32)
    for k in range(_K):
        keep = (g > vprev[None, :]) | ((g == vprev[None, :])
                                       & (gio > iprev[None, :]))
        dv = jnp.where(keep, g, _BIG)
        m = jnp.min(dv, axis=0)
        ii = jnp.min(jnp.where(dv == m[None, :], gio, _IBIG), axis=0)
        vprev, iprev = m, ii
        grp_ref[k, :] = ii


def _topk(a2p, b1m):
    B, n2, _ = a2p.shape
    n1 = b1m.shape[2]
    return pl.pallas_call(
        _topk_body,
        grid=(B, n1 // _QT),
        in_specs=[
            pl.BlockSpec((None, n2, 8), lambda b, t: (b, 0, 0)),
            pl.BlockSpec((None, 8, _QT), lambda b, t: (b, 0, t)),
        ],
        out_specs=pl.BlockSpec((None, _K, _QT), lambda b, t: (b, 0, t)),
        out_shape=jax.ShapeDtypeStruct((B, _K, n1), jnp.int32),
        scratch_shapes=[pltpu.VMEM((n2, _QT), jnp.float32)],
    )(a2p, b1m)


def _refine(grp, kx, ky, kz, cn, qx, qy, qz, rn, B, n1, n2):
    K = _K
    wpb = _NWORK // B
    npb = n1 // wpb
    npacks = npb // 16
    mesh = plsc.VectorSubcoreMesh(core_axis_name="c", subcore_axis_name="s")

    @functools.partial(
        pl.kernel,
        out_type=jax.ShapeDtypeStruct((B * K * n1,), jnp.int32),
        mesh=mesh,
        compiler_params=pltpu.CompilerParams(needs_layout_passes=False),
        scratch_types=[
            pltpu.VMEM((n2,), jnp.float32),
            pltpu.VMEM((n2,), jnp.float32),
            pltpu.VMEM((n2,), jnp.float32),
            pltpu.VMEM((n2,), jnp.float32),
            pltpu.VMEM((npb,), jnp.float32),
            pltpu.VMEM((npb,), jnp.float32),
            pltpu.VMEM((npb,), jnp.float32),
            pltpu.VMEM((npb,), jnp.float32),
            pltpu.VMEM((K * npb,), jnp.float32),
            pltpu.VMEM((K * _GS * 16,), jnp.float32),
            pltpu.VMEM((K * 16,), jnp.float32),
            pltpu.VMEM((K * 16,), jnp.int32),
            pltpu.VMEM((K * npb,), jnp.int32),
        ],
    )
    def rk(grp_h, kx_h, ky_h, kz_h, cn_h, qx_h, qy_h, qz_h, rn_h, out_h,
           kx_v, ky_v, kz_v, cn_v, qx_v, qy_v, qz_v, rn_v, grp_v,
           cand_d, M_d, M_i, o_w):
        wid = lax.axis_index("s") * _SC_CORES + lax.axis_index("c")
        b = wid // wpb
        n0 = (wid % wpb) * npb
        pltpu.sync_copy(kx_h.at[pl.ds(b * n2, n2)], kx_v)
        pltpu.sync_copy(ky_h.at[pl.ds(b * n2, n2)], ky_v)
        pltpu.sync_copy(kz_h.at[pl.ds(b * n2, n2)], kz_v)
        pltpu.sync_copy(cn_h.at[pl.ds(b * n2, n2)], cn_v)
        pltpu.sync_copy(qx_h.at[pl.ds(b * n1 + n0, npb)], qx_v)
        pltpu.sync_copy(qy_h.at[pl.ds(b * n1 + n0, npb)], qy_v)
        pltpu.sync_copy(qz_h.at[pl.ds(b * n1 + n0, npb)], qz_v)
        pltpu.sync_copy(rn_h.at[pl.ds(b * n1 + n0, npb)], rn_v)
        for t in range(K):
            pltpu.sync_copy(grp_h.at[pl.ds((b * K + t) * n1 + n0, npb)],
                            grp_v.at[pl.ds(t * npb, npb)])
        lanes = lax.iota(jnp.int32, 16)
        zeros = jnp.zeros((16,), jnp.int32)

        def pack_step(p, carry):
            qxl = qx_v[pl.ds(p * 16, 16)]
            qyl = qy_v[pl.ds(p * 16, 16)]
            qzl = qz_v[pl.ds(p * 16, 16)]
            rnl = rn_v[pl.ds(p * 16, 16)]
            col = p * 16 + lanes

            def grp_step(t, c2):
                gvec = plsc.load_gather(grp_v, [t * npb + col]).astype(jnp.int32)
                bd = jnp.full((16,), _BIG, jnp.float32)
                bi = jnp.full((16,), _IBIG, jnp.int32)
                for e in range(_GS):
                    kvec = gvec * _GS + e
                    dx = plsc.load_gather(kx_v, [kvec])
                    dy = plsc.load_gather(ky_v, [kvec])
                    dz = plsc.load_gather(kz_v, [kvec])
                    cc = plsc.load_gather(cn_v, [kvec])
                    d0 = (dx * qxl + dy * qyl) + dz * qzl
                    d = (d0 + rnl) + cc
                    plsc.store_scatter(cand_d, [(t * _GS + e) * 16 + lanes], d)
                    lt = (d < bd) | ((d == bd) & (kvec < bi))
                    bd = jnp.where(lt, d, bd)
                    bi = jnp.where(lt, kvec, bi)
                plsc.store_scatter(M_d, [t * 16 + lanes], bd)
                plsc.store_scatter(M_i, [t * 16 + lanes], bi)
                return c2

            lax.fori_loop(0, K, grp_step, 0)

            def sel_step(k, c2):
                cv = jnp.full((16,), _BIG, jnp.float32)
                ci = jnp.full((16,), _IBIG, jnp.int32)
                wv = jnp.zeros((16,), jnp.int32)
                for rb in range(K):
                    md = M_d[pl.ds(rb * 16, 16)]
                    mi = M_i[pl.ds(rb * 16, 16)]
                    lt = (md < cv) | ((md == cv) & (mi < ci))
                    cv = jnp.where(lt, md, cv)
                    ci = jnp.where(lt, mi, ci)
                    wv = jnp.where(lt, zeros + rb, wv)
                plsc.store_scatter(o_w, [k * npb + col], ci)
                gw = plsc.load_gather(grp_v, [wv * npb + col]).astype(jnp.int32)
                nd = jnp.full((16,), _BIG, jnp.float32)
                ni = jnp.full((16,), _IBIG, jnp.int32)
                for e in range(_GS):
                    dd = plsc.load_gather(cand_d, [(wv * _GS + e) * 16 + lanes])
                    ii = gw * _GS + e
                    valid = (dd > cv) | ((dd == cv) & (ii > ci))
                    dd2 = jnp.where(valid, dd, _BIG)
                    ii2 = jnp.where(valid, ii, _IBIG)
                    lt = (dd2 < nd) | ((dd2 == nd) & (ii2 < ni))
                    nd = jnp.where(lt, dd2, nd)
                    ni = jnp.where(lt, ii2, ni)
                plsc.store_scatter(M_d, [wv * 16 + lanes], nd)
                plsc.store_scatter(M_i, [wv * 16 + lanes], ni)
                return c2

            lax.fori_loop(0, K, sel_step, 0)
            return carry

        lax.fori_loop(0, npacks, pack_step, 0)
        for t in range(K):
            pltpu.sync_copy(o_w.at[pl.ds(t * npb, npb)],
                            out_h.at[pl.ds((b * K + t) * n1 + n0, npb)])

    return rk(grp, kx, ky, kz, cn, qx, qy, qz, rn)


def _pre_body(f1_ref, f2_ref, p1_ref, p2_ref, w1a_ref, w1b_ref, w1c_ref,
              b1_ref, q_ref, g2_ref):
    dn = (((0,), (1,)), ((), ()))
    g2 = (lax.dot_general(f2_ref[:, :], w1b_ref[:, :], dn, **_HP)
          + lax.dot_general(p2_ref[:, :], w1c_ref[:, :], dn, **_HP))
    g2_ref[:, 0:64] = g2
    g2_ref[:, 64:128] = jnp.zeros_like(g2)
    q_ref[:, :] = (lax.dot_general(f1_ref[:, :], w1a_ref[:, :], dn, **_HP)
                   - lax.dot_general(p1_ref[:, :], w1c_ref[:, :], dn, **_HP)
                   + b1_ref[:, :])


def _pre(feat1, feat2, pc1p, pc2p, w1a, w1b, w1c, b1m):
    B, D, n1 = feat1.shape
    n2 = feat2.shape[2]
    return pl.pallas_call(
        _pre_body,
        grid=(B, n1 // _NT),
        in_specs=[
            pl.BlockSpec((None, D, _NT), lambda b, t: (b, 0, t)),
            pl.BlockSpec((None, D, _NT), lambda b, t: (b, 0, t)),
            pl.BlockSpec((None, 8, _NT), lambda b, t: (b, 0, t)),
            pl.BlockSpec((None, 8, _NT), lambda b, t: (b, 0, t)),
            pl.BlockSpec((D, D), lambda b, t: (0, 0)),
            pl.BlockSpec((D, D), lambda b, t: (0, 0)),
            pl.BlockSpec((D, 8), lambda b, t: (0, 0)),
            pl.BlockSpec((1, D), lambda b, t: (0, 0)),
        ],
        out_specs=[
            pl.BlockSpec((None, _NT, D), lambda b, t: (b, t, 0)),
            pl.BlockSpec((None, _NT, 2 * D), lambda b, t: (b, t, 0)),
        ],
        out_shape=[jax.ShapeDtypeStruct((B, n1, D), jnp.float32),
                   jax.ShapeDtypeStruct((B, n2, 2 * D), jnp.float32)],
    )(feat1, feat2, pc1p, pc2p, w1a, w1b, w1c, b1m)


def _gather(table, fidx):
    rows, d = fidx.shape[0], table.shape[1]
    per_w = rows // _NWORK
    nch = per_w // _RCH
    mesh = plsc.VectorSubcoreMesh(core_axis_name="c", subcore_axis_name="s")

    @functools.partial(
        pl.kernel,
        out_type=jax.ShapeDtypeStruct((rows, d), jnp.float32),
        mesh=mesh,
        scratch_types=[
            pltpu.VMEM((_RCH,), jnp.int32),
            pltpu.VMEM((_RCH, d), jnp.float32),
            pltpu.SemaphoreType.DMA,
        ],
    )
    def gk(table_hbm, idx_hbm, out_hbm, idx_v, rows_v, sem):
        wid = lax.axis_index("s") * _SC_CORES + lax.axis_index("c")

        def step(j, carry):
            base = wid * per_w + j * _RCH
            pltpu.sync_copy(idx_hbm.at[pl.ds(base, _RCH)], idx_v)
            pltpu.async_copy(table_hbm.at[idx_v], rows_v, sem).wait()
            pltpu.sync_copy(rows_v, out_hbm.at[pl.ds(base, _RCH)])
            return carry

        lax.fori_loop(0, nch, step, 0)

    return gk(table, fidx)


def _mlp_body(g_ref, q_ref, w2_ref, b2_ref, o_ref):
    q = q_ref[:, :]
    acc = jnp.full(q.shape, -_BIG, jnp.float32)
    dn = (((1,), (1,)), ((), ()))
    for k in range(_K):
        h = _leaky(g_ref[k, :, 0:64] + q)
        y = _leaky(lax.dot_general(h, w2_ref[:, :], dn, **_HP) + b2_ref[:, :])
        acc = jnp.maximum(acc, y)
    o_ref[:, :] = acc


def _mlp(gath, q, W2, b2m):
    B, K, n1, Dp = gath.shape
    D = q.shape[2]
    return pl.pallas_call(
        _mlp_body,
        grid=(B, n1 // _NT),
        in_specs=[
            pl.BlockSpec((None, K, _NT, Dp), lambda b, t: (b, 0, t, 0)),
            pl.BlockSpec((None, _NT, D), lambda b, t: (b, t, 0)),
            pl.BlockSpec((D, D), lambda b, t: (0, 0)),
            pl.BlockSpec((1, D), lambda b, t: (0, 0)),
        ],
        out_specs=pl.BlockSpec((None, _NT, D), lambda b, t: (b, t, 0)),
        out_shape=jax.ShapeDtypeStruct((B, n1, D), jnp.float32),
    )(gath, q, W2, b2m)


def _bf(x):
    u = lax.bitcast_convert_type(x, jnp.uint32)
    r = (u + jnp.uint32(0x7FFF) + ((u >> 16) & jnp.uint32(1))) \
        & jnp.uint32(0xFFFF0000)
    return lax.bitcast_convert_type(r, jnp.float32)


def kernel(pc1, pc2, feat1, feat2, W1, b1, W2, b2):
    B, _, n1 = pc1.shape
    n2 = pc2.shape[2]
    D = feat1.shape[1]
    f32 = jnp.float32

    xyz2t = jnp.transpose(pc2, (0, 2, 1))
    a2 = jnp.concatenate(
        [-2.0 * xyz2t,
         jnp.sum(xyz2t ** 2, axis=-1, keepdims=True),
         jnp.zeros((B, n2, 4), f32)], axis=-1)
    a2p = a2.reshape(B, _NG, _GS, 8).transpose(0, 2, 1, 3).reshape(B, n2, 8)
    b1m = jnp.concatenate([pc1, jnp.zeros((B, 5, n1), f32)], axis=1)
    grp = _topk(a2p, b1m)

    kx, ky, kz = _bf(-2.0 * pc2[:, 0]), _bf(-2.0 * pc2[:, 1]), _bf(-2.0 * pc2[:, 2])
    cn2 = jnp.sum(xyz2t ** 2, axis=-1)
    qx, qy, qz = _bf(pc1[:, 0]), _bf(pc1[:, 1]), _bf(pc1[:, 2])
    rn1 = (pc1[:, 0] * pc1[:, 0] + pc1[:, 1] * pc1[:, 1]) + pc1[:, 2] * pc1[:, 2]
    knn_f = _refine(grp.astype(f32).reshape(-1), kx.reshape(-1),
                    ky.reshape(-1), kz.reshape(-1), cn2.reshape(-1),
                    qx.reshape(-1), qy.reshape(-1), qz.reshape(-1),
                    rn1.reshape(-1), B, n1, n2)
    knn = knn_f.reshape(B, _K, n1)

    pc1p = jnp.concatenate([pc1, jnp.zeros((B, 5, n1), f32)], axis=1)
    pc2p = jnp.concatenate([pc2, jnp.zeros((B, 5, n2), f32)], axis=1)
    w1a = W1[:, :D]
    w1b = W1[:, D:2 * D]
    w1c = jnp.concatenate([W1[:, 2 * D:], jnp.zeros((D, 5), f32)], axis=1)
    q, g2 = _pre(feat1, feat2, pc1p, pc2p, w1a, w1b, w1c, b1.reshape(1, D))

    fidx = (knn + (jnp.arange(B, dtype=jnp.int32) * n2)[:, None, None]
            ).reshape(-1)
    gath = _gather(g2.reshape(B * n2, 2 * D), fidx).reshape(B, _K, n1, 2 * D)
    out = _mlp(gath, q, W2, b2.reshape(1, D))
    return jnp.transpose(out, (0, 2, 1))
